# Initial kernel scaffold; baseline (speedup 1.0000x reference)
import jax
import jax.numpy as jnp
from jax.experimental import pallas as pl


def kernel(x, edge_index, batch, W_gat, att_src, att_dst, b_gat, W1, b1, W2, b2, Wf, bf):
    raise NotImplementedError("write your pallas kernel here")


# speedup vs baseline: 1.1581x; 1.1581x over previous
import functools

import jax
import jax.numpy as jnp
from jax import lax
from jax.experimental import pallas as pl
from jax.experimental.pallas import tpu as pltpu
from jax.experimental.pallas import tpu_sc as plsc

N = 10000
E = 320000
IN_C = 128
OUT_C = 128
HEADS = 4
G = 64

NC = 2
NS = 16
K = 80


def _gin_agg_sc(src, dst, q):
    mesh = plsc.VectorSubcoreMesh(core_axis_name="c", subcore_axis_name="s")
    ew = E // (NC * NS)
    rpt = 632

    @functools.partial(
        pl.kernel,
        mesh=mesh,
        out_type=jax.ShapeDtypeStruct((NC, N, 128), jnp.float32),
        scratch_types=[
            pltpu.VMEM((K,), jnp.int32),
            pltpu.VMEM((K,), jnp.int32),
            pltpu.VMEM((K, 128), jnp.float32),
            pltpu.VMEM((8, 128), jnp.float32),
            pltpu.VMEM_SHARED((N, 128), jnp.float32),
            pltpu.SemaphoreType.DMA,
        ],
    )
    def k(src_hbm, dst_hbm, q_hbm, out_hbm, sidx, didx, rows, zbuf, acc, sem):
        c = lax.axis_index("c")
        s = lax.axis_index("s")

        def zrow(i, _):
            def zlane(j, _):
                zbuf[i, pl.ds(j * 16, 16)] = jnp.zeros((16,), jnp.float32)
                return 0
            return lax.fori_loop(0, 8, zlane, 0)
        lax.fori_loop(0, 8, zrow, 0)
        row0 = s * rpt
        nz = jnp.where(s == NS - 1, (N - (NS - 1) * rpt) // 8, rpt // 8)

        def zchunk(i, _):
            pltpu.sync_copy(zbuf, acc.at[pl.ds(row0 + i * 8, 8)])
            return 0
        lax.fori_loop(0, nz, zchunk, 0)
        plsc.subcore_barrier()

        base = (c * NS + s) * ew

        def body(g, _):
            e0 = base + g * K
            pltpu.sync_copy(src_hbm.at[pl.ds(e0, K)], sidx)
            pltpu.sync_copy(dst_hbm.at[pl.ds(e0, K)], didx)
            pltpu.async_copy(q_hbm.at[sidx], rows, sem).wait()
            pltpu.sync_copy(rows, acc.at[didx], add=True)
            return 0
        lax.fori_loop(0, ew // K, body, 0)
        plsc.subcore_barrier()

        @pl.when(s < NS - 1)
        def _():
            pltpu.sync_copy(acc.at[pl.ds(row0, rpt)], out_hbm.at[c, pl.ds(row0, rpt)])

        @pl.when(s == NS - 1)
        def _():
            r0 = (NS - 1) * rpt
            pltpu.sync_copy(acc.at[pl.ds(r0, N - (NS - 1) * rpt)],
                            out_hbm.at[c, pl.ds(r0, N - (NS - 1) * rpt)])

    return k(src, dst, q)


def kernel(x, edge_index, batch, W_gat, att_src, att_dst, b_gat, W1, b1, W2, b2, Wf, bf):
    src, dst = edge_index[0], edge_index[1]
    h = (x @ W_gat).reshape(N, HEADS, OUT_C)
    a_src = jnp.sum(h * att_src, axis=-1)
    a_dst = jnp.sum(h * att_dst, axis=-1)

    loop = jnp.arange(N, dtype=src.dtype)
    src_sl = jnp.concatenate([src, loop])
    dst_sl = jnp.concatenate([dst, loop])
    alpha = a_src[src_sl] + a_dst[dst_sl]
    alpha = jax.nn.leaky_relu(alpha, negative_slope=0.2)
    ea = jnp.exp(alpha)
    denom = jax.ops.segment_sum(ea, dst_sl, num_segments=N)
    msg = h[src_sl] * ea[:, :, None]
    out = jax.ops.segment_sum(msg, dst_sl, num_segments=N)
    out = out / (denom[:, :, None] + 1e-16)
    out = out.reshape(N, HEADS * OUT_C) + b_gat
    out = jax.nn.relu(out)

    q = out @ W1
    agg_p = _gin_agg_sc(src, dst, q)
    g = jax.nn.relu(q + agg_p[0] + agg_p[1] + b1)
    g = g @ W2 + b2

    sums = jax.ops.segment_sum(g, batch, num_segments=G)
    counts = jax.ops.segment_sum(jnp.ones((N,), jnp.float32), batch, num_segments=G)
    pooled = sums / jnp.maximum(counts, 1.0)[:, None]
    return pooled @ Wf + bf


# speedup vs baseline: 17.0931x; 14.7601x over previous
import functools

import jax
import jax.numpy as jnp
from jax import lax
from jax.experimental import pallas as pl
from jax.experimental.pallas import tpu as pltpu
from jax.experimental.pallas import tpu_sc as plsc

N = 10000
E = 320000
IN_C = 128
OUT_C = 128
HEADS = 4
G = 64

NC = 2
NS = 16
K = 80
RPT = 632
DPT = 640


def _gin_agg_sc(src, dst, q):
    mesh = plsc.VectorSubcoreMesh(core_axis_name="c", subcore_axis_name="s")
    ew = E // (NC * NS)

    @functools.partial(
        pl.kernel,
        mesh=mesh,
        out_type=jax.ShapeDtypeStruct((NC, N, 128), jnp.float32),
        scratch_types=[
            pltpu.VMEM((K,), jnp.int32),
            pltpu.VMEM((K,), jnp.int32),
            pltpu.VMEM((K, 128), jnp.float32),
            pltpu.VMEM((8, 128), jnp.float32),
            pltpu.VMEM_SHARED((N, 128), jnp.float32),
            pltpu.SemaphoreType.DMA,
        ],
    )
    def k(src_hbm, dst_hbm, q_hbm, out_hbm, sidx, didx, rows, zbuf, acc, sem):
        c = lax.axis_index("c")
        s = lax.axis_index("s")

        def zrow(i, _):
            def zlane(j, _):
                zbuf[i, pl.ds(j * 16, 16)] = jnp.zeros((16,), jnp.float32)
                return 0
            return lax.fori_loop(0, 8, zlane, 0)
        lax.fori_loop(0, 8, zrow, 0)
        row0 = s * RPT
        nz = jnp.where(s == NS - 1, (N - (NS - 1) * RPT) // 8, RPT // 8)

        def zchunk(i, _):
            pltpu.sync_copy(zbuf, acc.at[pl.ds(row0 + i * 8, 8)])
            return 0
        lax.fori_loop(0, nz, zchunk, 0)
        plsc.subcore_barrier()

        base = (c * NS + s) * ew

        def body(g, _):
            e0 = base + g * K
            pltpu.sync_copy(src_hbm.at[pl.ds(e0, K)], sidx)
            pltpu.sync_copy(dst_hbm.at[pl.ds(e0, K)], didx)
            pltpu.async_copy(q_hbm.at[sidx], rows, sem).wait()
            pltpu.sync_copy(rows, acc.at[didx], add=True)
            return 0
        lax.fori_loop(0, ew // K, body, 0)
        plsc.subcore_barrier()

        @pl.when(s < NS - 1)
        def _():
            pltpu.sync_copy(acc.at[pl.ds(row0, RPT)], out_hbm.at[c, pl.ds(row0, RPT)])

        @pl.when(s == NS - 1)
        def _():
            r0 = (NS - 1) * RPT
            pltpu.sync_copy(acc.at[pl.ds(r0, N - (NS - 1) * RPT)],
                            out_hbm.at[c, pl.ds(r0, N - (NS - 1) * RPT)])

    return k(src, dst, q)


def _gat_edge_sc(src, dst, h4, asrcp, adstp):
    mesh = plsc.VectorSubcoreMesh(core_axis_name="c", subcore_axis_name="s")
    ew = E // (NC * NS)

    @functools.partial(
        pl.kernel,
        mesh=mesh,
        out_type=(
            jax.ShapeDtypeStruct((HEADS, NC, N, 128), jnp.float32),
            jax.ShapeDtypeStruct((NC, N, 128), jnp.float32),
            jax.ShapeDtypeStruct((E, 128), jnp.float32),
        ),
        scratch_types=[
            pltpu.VMEM((K,), jnp.int32),
            pltpu.VMEM((K,), jnp.int32),
            pltpu.VMEM((K, 128), jnp.float32),
            pltpu.VMEM((K, 128), jnp.float32),
            pltpu.VMEM((K, 128), jnp.float32),
            pltpu.VMEM((K, 128), jnp.float32),
            pltpu.VMEM((8, 128), jnp.float32),
            pltpu.VMEM_SHARED((N, 128), jnp.float32),
            pltpu.SemaphoreType.DMA,
            pltpu.SemaphoreType.DMA,
        ],
    )
    def k(src_hbm, dst_hbm, h4_hbm, asrc_hbm, adst_hbm,
          out_hbm, den_hbm, ea4_hbm,
          sidx, didx, rows, asbuf, adbuf, earows, zbuf, acc, sem1, sem2):
        c = lax.axis_index("c")
        s = lax.axis_index("s")
        zero16 = jnp.zeros((16,), jnp.float32)
        pt2 = jnp.full((16,), 0.2, jnp.float32)

        def zrow(i, _):
            def zlane(j, _):
                zbuf[i, pl.ds(j * 16, 16)] = zero16
                return 0
            return lax.fori_loop(0, 8, zlane, 0)
        lax.fori_loop(0, 8, zrow, 0)

        def zear(i, _):
            def zl(j, _):
                earows[i, pl.ds(j * 16, 16)] = zero16
                return 0
            return lax.fori_loop(0, 8, zl, 0)
        lax.fori_loop(0, K, zear, 0)

        row0 = s * RPT
        nz = jnp.where(s == NS - 1, (N - (NS - 1) * RPT) // 8, RPT // 8)
        base = (c * NS + s) * ew

        def zero_acc():
            def zchunk(i, _):
                pltpu.sync_copy(zbuf, acc.at[pl.ds(row0 + i * 8, 8)])
                return 0
            lax.fori_loop(0, nz, zchunk, 0)

        def readout(dst_ref):
            @pl.when(s < NS - 1)
            def _():
                pltpu.sync_copy(acc.at[pl.ds(row0, RPT)],
                                dst_ref.at[pl.ds(row0, RPT)])

            @pl.when(s == NS - 1)
            def _():
                r0 = (NS - 1) * RPT
                pltpu.sync_copy(acc.at[pl.ds(r0, N - (NS - 1) * RPT)],
                                dst_ref.at[pl.ds(r0, N - (NS - 1) * RPT)])

        zero_acc()
        plsc.subcore_barrier()

        def body0(g, _):
            e0 = base + g * K
            pltpu.sync_copy(src_hbm.at[pl.ds(e0, K)], sidx)
            pltpu.sync_copy(dst_hbm.at[pl.ds(e0, K)], didx)
            cp1 = pltpu.async_copy(asrc_hbm.at[sidx], asbuf, sem1)
            cp2 = pltpu.async_copy(adst_hbm.at[didx], adbuf, sem2)
            cp1.wait()
            cp2.wait()

            def ecomp(j, _):
                for hh in range(HEADS):
                    sl = pl.ds(hh * 32, 16)
                    al = asbuf[j, sl] + adbuf[j, sl]
                    al = jnp.where(al >= zero16, al, al * pt2)
                    earows[j, sl] = jnp.exp(al)
                return 0
            lax.fori_loop(0, K, ecomp, 0)
            pltpu.sync_copy(earows, ea4_hbm.at[pl.ds(e0, K)])
            pltpu.sync_copy(earows, acc.at[didx], add=True)
            return 0
        lax.fori_loop(0, ew // K, body0, 0)
        plsc.subcore_barrier()
        readout(den_hbm.at[c])
        plsc.subcore_barrier()

        for hh in range(HEADS):
            zero_acc()
            plsc.subcore_barrier()

            def bodyh(g, _):
                e0 = base + g * K
                pltpu.sync_copy(src_hbm.at[pl.ds(e0, K)], sidx)
                pltpu.sync_copy(dst_hbm.at[pl.ds(e0, K)], didx)
                cp = pltpu.async_copy(h4_hbm.at[hh].at[sidx], rows, sem1)
                pltpu.sync_copy(ea4_hbm.at[pl.ds(e0, K)], adbuf)
                cp.wait()

                def scale(j, _):
                    ea = adbuf[j, pl.ds(hh * 32, 16)]

                    def mul(cc, _):
                        sl = pl.ds(cc * 16, 16)
                        rows[j, sl] = rows[j, sl] * ea
                        return 0
                    lax.fori_loop(0, 8, mul, 0)
                    return 0
                lax.fori_loop(0, K, scale, 0)
                pltpu.sync_copy(rows, acc.at[didx], add=True)
                return 0
            lax.fori_loop(0, ew // K, bodyh, 0)
            plsc.subcore_barrier()
            readout(out_hbm.at[hh, c])
            plsc.subcore_barrier()

    return k(src, dst, h4, asrcp, adstp)


def kernel(x, edge_index, batch, W_gat, att_src, att_dst, b_gat, W1, b1, W2, b2, Wf, bf):
    src, dst = edge_index[0], edge_index[1]
    h = (x @ W_gat).reshape(N, HEADS, OUT_C)
    a_src = jnp.sum(h * att_src, axis=-1)
    a_dst = jnp.sum(h * att_dst, axis=-1)
    h4 = jnp.transpose(h, (1, 0, 2))

    asrcp = jnp.broadcast_to(a_src[:, :, None], (N, HEADS, 32)).reshape(N, 128)
    adstp = jnp.broadcast_to(a_dst[:, :, None], (N, HEADS, 32)).reshape(N, 128)
    out4, den, _ea4 = _gat_edge_sc(src, dst, h4, asrcp, adstp)
    num = out4[:, 0] + out4[:, 1]
    dall = den[0] + den[1]
    dsum = jnp.transpose(dall.reshape(N, HEADS, 32)[:, :, 0])

    ea_self = jnp.exp(jax.nn.leaky_relu(a_src + a_dst, 0.2))
    ea_self_t = jnp.transpose(ea_self)
    outh = (num + ea_self_t[:, :, None] * h4) / (
        (dsum + ea_self_t)[:, :, None] + 1e-16)
    out = jnp.transpose(outh, (1, 0, 2)).reshape(N, HEADS * OUT_C)
    out = jax.nn.relu(out + b_gat)

    q = out @ W1
    agg_p = _gin_agg_sc(src, dst, q)
    g = jax.nn.relu(q + agg_p[0] + agg_p[1] + b1)
    g = g @ W2 + b2

    sums = jax.ops.segment_sum(g, batch, num_segments=G)
    counts = jax.ops.segment_sum(jnp.ones((N,), jnp.float32), batch, num_segments=G)
    pooled = sums / jnp.maximum(counts, 1.0)[:, None]
    return pooled @ Wf + bf
